# Initial kernel scaffold; baseline (speedup 1.0000x reference)
import jax
import jax.numpy as jnp
from jax.experimental import pallas as pl


def kernel(query_emb, query_time, pool_emb, pool_time, lambda_decay):
    raise NotImplementedError("write your pallas kernel here")


# speedup vs baseline: 2.2132x; 2.2132x over previous
import functools

import jax
import jax.numpy as jnp
from jax.experimental import pallas as pl
from jax.experimental.pallas import tpu as pltpu

_K = 7
_W = 2048

_NEG_INF = float("-inf")
_I32_MAX = jnp.iinfo(jnp.int32).max


def _topk_kernel(q_ref, qt_ref, p_ref, pt_ref, lam_ref, out_ref,
                 run_v_ref, run_i_ref, *, n_pool, n_steps):
    t = pl.program_id(0)
    q = q_ref[...]
    qn = jnp.sqrt(jnp.sum(q * q, axis=1, keepdims=True))
    q = q / jnp.maximum(qn, 1e-8)

    p = p_ref[...]
    pn = jnp.sqrt(jnp.sum(p * p, axis=1, keepdims=True))
    p = p / jnp.maximum(pn, 1e-8)

    sim = jax.lax.dot_general(
        q, p, (((1,), (1,)), ((), ())), preferred_element_type=jnp.float32)

    qt = qt_ref[...]
    pt = pt_ref[...]
    lam = lam_ref[0, 0]
    tw = jnp.exp(-lam * jnp.abs(qt - pt))
    sim = sim * tw

    nq = sim.shape[0]
    col = jax.lax.broadcasted_iota(jnp.int32, (nq, _W), 1) + t * _W
    sim = jnp.where(col < n_pool, sim, _NEG_INF)

    @pl.when(t == 0)
    def _init():
        run_v_ref[...] = jnp.full(run_v_ref.shape, _NEG_INF, jnp.float32)
        run_i_ref[...] = jnp.zeros(run_i_ref.shape, jnp.int32)

    av = jnp.concatenate([sim, run_v_ref[...]], axis=1)
    ai = jnp.concatenate([col, run_i_ref[...]], axis=1)

    vals, idxs = [], []
    for _ in range(_K):
        m = jnp.max(av, axis=1, keepdims=True)
        wm = av == m
        wi = jnp.min(jnp.where(wm, ai, _I32_MAX), axis=1, keepdims=True)
        vals.append(m)
        idxs.append(wi)
        av = jnp.where(wm & (ai == wi), _NEG_INF, av)

    pad = run_v_ref.shape[1] - _K
    run_v_ref[...] = jnp.concatenate(
        vals + [jnp.full((nq, pad), _NEG_INF, jnp.float32)], axis=1)
    run_i_ref[...] = jnp.concatenate(
        idxs + [jnp.zeros((nq, pad), jnp.int32)], axis=1)

    @pl.when(t == n_steps - 1)
    def _emit():
        out_ref[...] = jnp.concatenate(
            idxs + [jnp.zeros((nq, 1), jnp.int32)], axis=1)


def kernel(query_emb, query_time, pool_emb, pool_time, lambda_decay):
    nq, d = query_emb.shape
    n_pool = pool_emb.shape[0]
    n_steps = -(-n_pool // _W)
    p_pad = n_steps * _W

    pool_p = jnp.pad(pool_emb, ((0, p_pad - n_pool), (0, 0)))
    pt_p = jnp.pad(pool_time, (0, p_pad - n_pool)).reshape(1, p_pad)
    qt = query_time.reshape(nq, 1)
    lam = jnp.reshape(lambda_decay, (1, 1)).astype(jnp.float32)

    out = pl.pallas_call(
        functools.partial(_topk_kernel, n_pool=n_pool, n_steps=n_steps),
        grid=(n_steps,),
        in_specs=[
            pl.BlockSpec((nq, d), lambda t: (0, 0)),
            pl.BlockSpec((nq, 1), lambda t: (0, 0)),
            pl.BlockSpec((_W, d), lambda t: (t, 0)),
            pl.BlockSpec((1, _W), lambda t: (0, t)),
            pl.BlockSpec((1, 1), lambda t: (0, 0)),
        ],
        out_specs=pl.BlockSpec((nq, 8), lambda t: (0, 0)),
        out_shape=jax.ShapeDtypeStruct((nq, 8), jnp.int32),
        scratch_shapes=[
            pltpu.VMEM((nq, 128), jnp.float32),
            pltpu.VMEM((nq, 128), jnp.int32),
        ],
    )(query_emb, qt, pool_p, pt_p, lam)
    return out[:, :_K]
